# baseline (speedup 1.0000x reference)
import functools

import jax
import jax.numpy as jnp
import numpy as np
from jax import lax
from jax.experimental import pallas as pl
from jax.experimental.pallas import tpu as pltpu
from jax.experimental.pallas import tpu_sc as plsc

N_LEVELS = 16
N_FEATS = 2
LOG2_T = 19
T = 2 ** LOG2_T
IN_DIM = 3
HASH_MASK = np.int32(T - 1)
P1 = np.int32(-1640531535)
P2 = np.int32(805459861)

N_PTS = 262144
NW = 32
PTS_PER_TILE = N_PTS // NW
C = 128
NCHUNK = PTS_PER_TILE // C
NI = N_LEVELS * 8
OUT_W = N_LEVELS * N_FEATS


def _splat_i(v):
    return jnp.full((16,), v, dtype=jnp.int32)


def _splat_f(v):
    return jnp.full((16,), v, dtype=jnp.float32)


def _make_encoder():
    mesh = plsc.VectorSubcoreMesh(core_axis_name="c", subcore_axis_name="s")

    @functools.partial(
        pl.kernel,
        mesh=mesh,
        compiler_params=pltpu.CompilerParams(needs_layout_passes=False,
                                             use_tc_tiling_on_sc=False),
        out_type=jax.ShapeDtypeStruct((N_PTS * OUT_W,), jnp.float32),
        scratch_types=[
            pltpu.VMEM((IN_DIM, C), jnp.float32),
            pltpu.VMEM((NI, C), jnp.int32),
            pltpu.VMEM((NI, C), jnp.int32),
            pltpu.VMEM((NI, C), jnp.float32),
            pltpu.VMEM((NI, C), jnp.float32),
            pltpu.VMEM((NI, C), jnp.float32),
            pltpu.VMEM((C * OUT_W,), jnp.float32),
            pltpu.SemaphoreType.DMA,
        ],
    )
    def enc_kernel(xnt_hbm, tab_hbm, out_hbm, xv, idx0, idx1, wb, f0b, f1b,
                   encv, sem):
        wid = lax.axis_index("s") * 2 + lax.axis_index("c")
        tile_base = wid * PTS_PER_TILE
        iota16 = lax.iota(jnp.int32, 16)

        def chunk_body(ci, _):
            base = tile_base + ci * C
            pltpu.sync_copy(xnt_hbm.at[:, pl.ds(base, C)], xv)

            def g_body(g, _):
                col = g * 16
                x0 = xv[0, pl.ds(col, 16)]
                x1 = xv[1, pl.ds(col, 16)]
                x2 = xv[2, pl.ds(col, 16)]

                def l_body(l, _):
                    res_f = jnp.full((16,), jnp.left_shift(16, l),
                                     dtype=jnp.int32).astype(jnp.float32)
                    pos0 = x0 * res_f
                    pos1 = x1 * res_f
                    pos2 = x2 * res_f
                    p0 = pos0.astype(jnp.int32)
                    p1 = pos1.astype(jnp.int32)
                    p2 = pos2.astype(jnp.int32)
                    w0 = pos0 - p0.astype(jnp.float32)
                    w1 = pos1 - p1.astype(jnp.float32)
                    w2 = pos2 - p2.astype(jnp.float32)
                    h1 = p1 * P1
                    h2 = p2 * P2
                    b0 = p0 + 1
                    b1 = h1 + P1
                    b2 = h2 + P2
                    txy = (p0 ^ h1, b0 ^ h1, p0 ^ b1, b0 ^ b1)
                    u0 = 1.0 - w0
                    u1 = 1.0 - w1
                    u2 = 1.0 - w2
                    axy = (u0 * u1, w0 * u1, u0 * w1, w0 * w1)
                    lbase = jnp.full((16,), jnp.left_shift(l, LOG2_T),
                                     dtype=jnp.int32)
                    for corner in range(8):
                        o2 = (corner >> 2) & 1
                        hz = b2 if o2 else h2
                        wz = w2 if o2 else u2
                        idx = ((txy[corner & 3] ^ hz) & HASH_MASK) | lbase
                        e0 = idx * 2
                        r = l * 8 + corner
                        idx0[r, pl.ds(col, 16)] = e0
                        idx1[r, pl.ds(col, 16)] = e0 + 1
                        wb[r, pl.ds(col, 16)] = axy[corner & 3] * wz
                    return ()

                lax.fori_loop(0, N_LEVELS, l_body, (), unroll=False)
                return ()

            lax.fori_loop(0, 8, g_body, (), unroll=False)

            def fire(j, _):
                pltpu.async_copy(tab_hbm.at[idx0.at[j]], f0b.at[j], sem)
                pltpu.async_copy(tab_hbm.at[idx1.at[j]], f1b.at[j], sem)
                return ()

            lax.fori_loop(0, NI, fire, (), unroll=False)

            def drain(j, _):
                pltpu.make_async_copy(tab_hbm.at[idx0.at[j]], f0b.at[j],
                                      sem).wait()
                pltpu.make_async_copy(tab_hbm.at[idx1.at[j]], f1b.at[j],
                                      sem).wait()
                return ()

            lax.fori_loop(0, NI, drain, (), unroll=False)

            def acc_body(k, _):
                l = k >> 3
                g = k & 7
                col = g * 16
                pidx = iota16 + col
                acc0 = _splat_f(0.0)
                acc1 = _splat_f(0.0)
                for corner in range(8):
                    r = l * 8 + corner
                    w = wb[r, pl.ds(col, 16)]
                    acc0 = acc0 + f0b[r, pl.ds(col, 16)] * w
                    acc1 = acc1 + f1b[r, pl.ds(col, 16)] * w
                dest0 = pidx * OUT_W + l * 2
                plsc.store_scatter(encv, [dest0], acc0)
                plsc.store_scatter(encv, [dest0 + 1], acc1)
                return ()

            lax.fori_loop(0, N_LEVELS * 8, acc_body, (), unroll=False)

            pltpu.sync_copy(encv, out_hbm.at[pl.ds(base * OUT_W, C * OUT_W)])
            return ()

        lax.fori_loop(0, NCHUNK, chunk_body, (), unroll=False)

    return enc_kernel


_ENCODER = _make_encoder()


def kernel(x, table, bounding_box):
    bb_min = bounding_box[:3]
    bb_max = bounding_box[3:]
    xn = (x - bb_min) / (bb_max - bb_min)
    mask = ((xn > 0) & (xn < 1)).all(axis=-1)
    xnt = xn.T
    tab1d = table.reshape(N_LEVELS * T * N_FEATS)
    enc = _ENCODER(xnt, tab1d).reshape(N_PTS, OUT_W)
    return (enc, mask)


# speedup vs baseline: 1.0076x; 1.0076x over previous
import functools

import jax
import jax.numpy as jnp
import numpy as np
from jax import lax
from jax.experimental import pallas as pl
from jax.experimental.pallas import tpu as pltpu
from jax.experimental.pallas import tpu_sc as plsc

N_LEVELS = 16
N_FEATS = 2
LOG2_T = 19
T = 2 ** LOG2_T
IN_DIM = 3
HASH_MASK = np.int32(T - 1)
P1 = np.int32(-1640531535)
P2 = np.int32(805459861)

N_PTS = 262144
NW = 32
C = 64
NG = C // 16
NI = N_LEVELS * 8
CHUNK_IDX = NI * C
NELEM = CHUNK_IDX * 2
NDMA = NELEM // 128
OUT_W = N_LEVELS * N_FEATS


def _splat_i(v):
    return jnp.full((16,), v, dtype=jnp.int32)


def _splat_f(v):
    return jnp.full((16,), v, dtype=jnp.float32)


def _make_encoder(n_pts=N_PTS, interpret=False):
    pts_per_tile = n_pts // NW
    nchunk = pts_per_tile // C
    mesh = plsc.VectorSubcoreMesh(core_axis_name="c", subcore_axis_name="s")

    @functools.partial(
        pl.kernel,
        mesh=mesh,
        compiler_params=pltpu.CompilerParams(needs_layout_passes=False,
                                             use_tc_tiling_on_sc=False),
        interpret=interpret,
        out_type=jax.ShapeDtypeStruct((n_pts * OUT_W,), jnp.float32),
        scratch_types=[
            pltpu.VMEM((IN_DIM, C), jnp.float32),
            pltpu.VMEM((NDMA, 128), jnp.int32),
            pltpu.VMEM((NDMA, 128), jnp.int32),
            pltpu.VMEM((CHUNK_IDX,), jnp.float32),
            pltpu.VMEM((CHUNK_IDX,), jnp.float32),
            pltpu.VMEM((NELEM,), jnp.float32),
            pltpu.VMEM((NELEM,), jnp.float32),
            pltpu.VMEM((C * OUT_W,), jnp.float32),
            pltpu.SemaphoreType.DMA,
            pltpu.SemaphoreType.DMA,
        ],
    )
    def enc_kernel(xnt_hbm, tab_hbm, out_hbm, xv, idx0, idx1, wb0, wb1,
                   fb0, fb1, encv, sem0, sem1):
        wid = lax.axis_index("s") * 2 + lax.axis_index("c")
        tile_base = wid * pts_per_tile
        iota16 = lax.iota(jnp.int32, 16)
        iota2 = iota16 * 2

        def phase_a(ci, idxb, wb):
            base = tile_base + ci * C
            pltpu.sync_copy(xnt_hbm.at[:, pl.ds(base, C)], xv)

            def g_body(g, _):
                col = g * 16
                x0 = xv[0, pl.ds(col, 16)]
                x1 = xv[1, pl.ds(col, 16)]
                x2 = xv[2, pl.ds(col, 16)]

                def l_body(l, _):
                    res_f = jnp.full((16,), jnp.left_shift(16, l),
                                     dtype=jnp.int32).astype(jnp.float32)
                    pos0 = x0 * res_f
                    pos1 = x1 * res_f
                    pos2 = x2 * res_f
                    p0 = pos0.astype(jnp.int32)
                    p1 = pos1.astype(jnp.int32)
                    p2 = pos2.astype(jnp.int32)
                    w0 = pos0 - p0.astype(jnp.float32)
                    w1 = pos1 - p1.astype(jnp.float32)
                    w2 = pos2 - p2.astype(jnp.float32)
                    h1 = p1 * P1
                    h2 = p2 * P2
                    b0 = p0 + 1
                    b1 = h1 + P1
                    b2 = h2 + P2
                    txy = (p0 ^ h1, b0 ^ h1, p0 ^ b1, b0 ^ b1)
                    u0 = 1.0 - w0
                    u1 = 1.0 - w1
                    u2 = 1.0 - w2
                    axy = (u0 * u1, w0 * u1, u0 * w1, w0 * w1)
                    lbase = jnp.full((16,), jnp.left_shift(l, LOG2_T),
                                     dtype=jnp.int32)
                    rbase = l * 8 * C + col
                    for corner in range(8):
                        o2 = (corner >> 2) & 1
                        hz = b2 if o2 else h2
                        wz = w2 if o2 else u2
                        idx = ((txy[corner & 3] ^ hz) & HASH_MASK) | lbase
                        e0 = idx * 2
                        flat = rbase + corner * C
                        flat2 = flat * 2
                        q = lax.shift_right_logical(flat2, 7)
                        fcol = jnp.bitwise_and(flat2, 127)
                        cole = iota2 + fcol
                        plsc.store_scatter(idxb, [_splat_i(q), cole], e0)
                        plsc.store_scatter(idxb, [_splat_i(q), cole + 1],
                                           e0 + 1)
                        wb[pl.ds(flat, 16)] = axy[corner & 3] * wz
                    return ()

                lax.fori_loop(0, N_LEVELS, l_body, (), unroll=False)
                return ()

            lax.fori_loop(0, NG, g_body, (), unroll=False)

        def fire(idxb, fb, sem):
            def f_body(j, _):
                pltpu.async_copy(tab_hbm.at[idxb.at[j]],
                                 fb.at[pl.ds(j * 128, 128)], sem)
                return ()

            lax.fori_loop(0, NDMA, f_body, (), unroll=False)

        def drain(idxb, fb, sem):
            def d_body(j, _):
                pltpu.make_async_copy(tab_hbm.at[idxb.at[j]],
                                      fb.at[pl.ds(j * 128, 128)],
                                      sem).wait()
                return ()

            lax.fori_loop(0, NDMA, d_body, (), unroll=False)

        def phase_b(ci, wb, fb):
            def acc_body(k, _):
                l = lax.shift_right_logical(k, 2)
                g = jnp.bitwise_and(k, NG - 1)
                col = g * 16
                pidx = iota16 + col
                rbase = l * 8 * C + col
                acc0 = _splat_f(0.0)
                acc1 = _splat_f(0.0)
                for corner in range(8):
                    flat = rbase + corner * C
                    w = wb[pl.ds(flat, 16)]
                    fvec2 = iota2 + flat * 2
                    acc0 = acc0 + plsc.load_gather(fb, [fvec2]) * w
                    acc1 = acc1 + plsc.load_gather(fb, [fvec2 + 1]) * w
                dest0 = pidx * OUT_W + l * 2
                plsc.store_scatter(encv, [dest0], acc0)
                plsc.store_scatter(encv, [dest0 + 1], acc1)
                return ()

            lax.fori_loop(0, N_LEVELS * NG, acc_body, (), unroll=False)
            base = tile_base + ci * C
            pltpu.sync_copy(encv, out_hbm.at[pl.ds(base * OUT_W, C * OUT_W)])

        phase_a(0, idx0, wb0)
        fire(idx0, fb0, sem0)

        def pipe_body(k, _):
            ca = 2 * k
            phase_a(ca + 1, idx1, wb1)
            fire(idx1, fb1, sem1)
            drain(idx0, fb0, sem0)
            phase_b(ca, wb0, fb0)

            @pl.when(k < nchunk // 2 - 1)
            def _():
                phase_a(ca + 2, idx0, wb0)
                fire(idx0, fb0, sem0)

            drain(idx1, fb1, sem1)
            phase_b(ca + 1, wb1, fb1)
            return ()

        lax.fori_loop(0, nchunk // 2, pipe_body, (), unroll=False)

    return enc_kernel


_ENCODER = _make_encoder()


def kernel(x, table, bounding_box):
    bb_min = bounding_box[:3]
    bb_max = bounding_box[3:]
    xn = (x - bb_min) / (bb_max - bb_min)
    mask = ((xn > 0) & (xn < 1)).all(axis=-1)
    xnt = xn.T
    tab1d = table.reshape(N_LEVELS * T * N_FEATS)
    enc = _ENCODER(xnt, tab1d).reshape(N_PTS, OUT_W)
    return (enc, mask)


# speedup vs baseline: 1.0085x; 1.0009x over previous
import functools

import jax
import jax.numpy as jnp
import numpy as np
from jax import lax
from jax.experimental import pallas as pl
from jax.experimental.pallas import tpu as pltpu
from jax.experimental.pallas import tpu_sc as plsc

N_LEVELS = 16
N_FEATS = 2
LOG2_T = 19
T = 2 ** LOG2_T
IN_DIM = 3
HASH_MASK = np.int32(T - 1)
P1 = np.int32(-1640531535)
P2 = np.int32(805459861)

N_PTS = 262144
NW = 32
C = 64
NG = C // 16
NI = N_LEVELS * 8
CHUNK_IDX = NI * C
NELEM = CHUNK_IDX * 2
NDMA = NELEM // 128
OUT_W = N_LEVELS * N_FEATS


def _splat_i(v):
    return jnp.full((16,), v, dtype=jnp.int32)


def _splat_f(v):
    return jnp.full((16,), v, dtype=jnp.float32)


def _make_encoder(n_pts=N_PTS, interpret=False):
    pts_per_tile = n_pts // NW
    nchunk = pts_per_tile // C
    mesh = plsc.VectorSubcoreMesh(core_axis_name="c", subcore_axis_name="s")

    @functools.partial(
        pl.kernel,
        mesh=mesh,
        compiler_params=pltpu.CompilerParams(needs_layout_passes=False,
                                             use_tc_tiling_on_sc=False),
        interpret=interpret,
        out_type=jax.ShapeDtypeStruct((n_pts * OUT_W,), jnp.float32),
        scratch_types=[
            pltpu.VMEM((IN_DIM, C), jnp.float32),
            pltpu.VMEM((NELEM,), jnp.int32),
            pltpu.VMEM((NELEM,), jnp.int32),
            pltpu.VMEM((CHUNK_IDX,), jnp.float32),
            pltpu.VMEM((CHUNK_IDX,), jnp.float32),
            pltpu.VMEM((NELEM,), jnp.float32),
            pltpu.VMEM((NELEM,), jnp.float32),
            pltpu.VMEM((C * OUT_W,), jnp.float32),
            pltpu.SemaphoreType.DMA,
            pltpu.SemaphoreType.DMA,
        ],
    )
    def enc_kernel(xnt_hbm, tab_hbm, out_hbm, xv, idx0, idx1, wb0, wb1,
                   fb0, fb1, encv, sem0, sem1):
        wid = lax.axis_index("s") * 2 + lax.axis_index("c")
        tile_base = wid * pts_per_tile
        iota16 = lax.iota(jnp.int32, 16)
        iota2 = iota16 * 2

        def phase_a(ci, idxb, wb):
            base = tile_base + ci * C
            pltpu.sync_copy(xnt_hbm.at[:, pl.ds(base, C)], xv)

            def g_body(g, _):
                col = g * 16
                x0 = xv[0, pl.ds(col, 16)]
                x1 = xv[1, pl.ds(col, 16)]
                x2 = xv[2, pl.ds(col, 16)]

                def l_body(l, _):
                    res_f = jnp.full((16,), jnp.left_shift(16, l),
                                     dtype=jnp.int32).astype(jnp.float32)
                    pos0 = x0 * res_f
                    pos1 = x1 * res_f
                    pos2 = x2 * res_f
                    p0 = pos0.astype(jnp.int32)
                    p1 = pos1.astype(jnp.int32)
                    p2 = pos2.astype(jnp.int32)
                    w0 = pos0 - p0.astype(jnp.float32)
                    w1 = pos1 - p1.astype(jnp.float32)
                    w2 = pos2 - p2.astype(jnp.float32)
                    h1 = p1 * P1
                    h2 = p2 * P2
                    b0 = p0 + 1
                    b1 = h1 + P1
                    b2 = h2 + P2
                    txy = (p0 ^ h1, b0 ^ h1, p0 ^ b1, b0 ^ b1)
                    u0 = 1.0 - w0
                    u1 = 1.0 - w1
                    u2 = 1.0 - w2
                    axy = (u0 * u1, w0 * u1, u0 * w1, w0 * w1)
                    lbase = jnp.full((16,), jnp.left_shift(l, LOG2_T),
                                     dtype=jnp.int32)
                    rbase = l * 8 * C + col
                    for corner in range(8):
                        o2 = (corner >> 2) & 1
                        hz = b2 if o2 else h2
                        wz = w2 if o2 else u2
                        idx = ((txy[corner & 3] ^ hz) & HASH_MASK) | lbase
                        e0 = idx * 2
                        flat = rbase + corner * C
                        cole = iota2 + flat * 2
                        plsc.store_scatter(idxb, [cole], e0)
                        plsc.store_scatter(idxb, [cole + 1], e0 + 1)
                        wb[pl.ds(flat, 16)] = axy[corner & 3] * wz
                    return ()

                lax.fori_loop(0, N_LEVELS, l_body, (), unroll=False)
                return ()

            lax.fori_loop(0, NG, g_body, (), unroll=False)

        def fire(idxb, fb, sem):
            pltpu.async_copy(tab_hbm.at[idxb], fb, sem)

        def drain(idxb, fb, sem):
            pltpu.make_async_copy(tab_hbm.at[idxb], fb, sem).wait()

        def phase_b(ci, wb, fb):
            def acc_body(k, _):
                l = lax.shift_right_logical(k, 2)
                g = jnp.bitwise_and(k, NG - 1)
                col = g * 16
                pidx = iota16 + col
                rbase = l * 8 * C + col
                acc0 = _splat_f(0.0)
                acc1 = _splat_f(0.0)
                for corner in range(8):
                    flat = rbase + corner * C
                    w = wb[pl.ds(flat, 16)]
                    fvec2 = iota2 + flat * 2
                    acc0 = acc0 + plsc.load_gather(fb, [fvec2]) * w
                    acc1 = acc1 + plsc.load_gather(fb, [fvec2 + 1]) * w
                dest0 = pidx * OUT_W + l * 2
                plsc.store_scatter(encv, [dest0], acc0)
                plsc.store_scatter(encv, [dest0 + 1], acc1)
                return ()

            lax.fori_loop(0, N_LEVELS * NG, acc_body, (), unroll=False)
            base = tile_base + ci * C
            pltpu.sync_copy(encv, out_hbm.at[pl.ds(base * OUT_W, C * OUT_W)])

        phase_a(0, idx0, wb0)
        fire(idx0, fb0, sem0)

        def pipe_body(k, _):
            ca = 2 * k
            phase_a(ca + 1, idx1, wb1)
            fire(idx1, fb1, sem1)
            drain(idx0, fb0, sem0)
            phase_b(ca, wb0, fb0)

            @pl.when(k < nchunk // 2 - 1)
            def _():
                phase_a(ca + 2, idx0, wb0)
                fire(idx0, fb0, sem0)

            drain(idx1, fb1, sem1)
            phase_b(ca + 1, wb1, fb1)
            return ()

        lax.fori_loop(0, nchunk // 2, pipe_body, (), unroll=False)

    return enc_kernel


_ENCODER = _make_encoder()


def kernel(x, table, bounding_box):
    bb_min = bounding_box[:3]
    bb_max = bounding_box[3:]
    xn = (x - bb_min) / (bb_max - bb_min)
    mask = ((xn > 0) & (xn < 1)).all(axis=-1)
    xnt = xn.T
    tab1d = table.reshape(N_LEVELS * T * N_FEATS)
    enc = _ENCODER(xnt, tab1d).reshape(N_PTS, OUT_W)
    return (enc, mask)


# speedup vs baseline: 4.9875x; 4.9455x over previous
import functools

import jax
import jax.numpy as jnp
import numpy as np
from jax import lax
from jax.experimental import pallas as pl
from jax.experimental.pallas import tpu as pltpu
from jax.experimental.pallas import tpu_sc as plsc

N_LEVELS = 16
N_FEATS = 2
LOG2_T = 19
T = 2 ** LOG2_T
IN_DIM = 3
HASH_MASK = np.int32(T - 1)
P1 = np.int32(-1640531535)
P2 = np.int32(805459861)
BLK_MASK = np.int32(-128)

N_PTS = 262144
NW = 32
C = 64
NG = C // 16
NI = N_LEVELS * 8
CHUNK_IDX = NI * C
NELEM = CHUNK_IDX * 2
NDMA = NELEM // 128
OUT_W = N_LEVELS * N_FEATS


def _splat_i(v):
    return jnp.full((16,), v, dtype=jnp.int32)


def _splat_f(v):
    return jnp.full((16,), v, dtype=jnp.float32)


def _make_encoder(n_pts=N_PTS, interpret=False):
    pts_per_tile = n_pts // NW
    nchunk = pts_per_tile // C
    mesh = plsc.VectorSubcoreMesh(core_axis_name="c", subcore_axis_name="s")

    @functools.partial(
        pl.kernel,
        mesh=mesh,
        compiler_params=pltpu.CompilerParams(needs_layout_passes=False,
                                             use_tc_tiling_on_sc=False),
        interpret=interpret,
        out_type=jax.ShapeDtypeStruct((n_pts * OUT_W,), jnp.float32),
        scratch_types=[
            pltpu.VMEM((IN_DIM, C), jnp.float32),
            pltpu.VMEM((NELEM,), jnp.int32),
            pltpu.VMEM((NELEM,), jnp.int32),
            pltpu.VMEM((CHUNK_IDX,), jnp.float32),
            pltpu.VMEM((CHUNK_IDX,), jnp.float32),
            pltpu.VMEM((NELEM,), jnp.float32),
            pltpu.VMEM((NELEM,), jnp.float32),
            pltpu.VMEM((C * OUT_W,), jnp.float32),
            pltpu.SemaphoreType.DMA,
            pltpu.SemaphoreType.DMA,
        ],
    )
    def enc_kernel(xnt_hbm, tab_hbm, out_hbm, xv, idx0, idx1, wb0, wb1,
                   fb0, fb1, encv, sem0, sem1):
        wid = lax.axis_index("s") * 2 + lax.axis_index("c")
        tile_base = wid * pts_per_tile
        iota16 = lax.iota(jnp.int32, 16)
        iota2 = iota16 * 2

        def phase_a(ci, idxb, wb):
            base = tile_base + ci * C
            pltpu.sync_copy(xnt_hbm.at[:, pl.ds(base, C)], xv)

            def g_body(g, _):
                col = g * 16
                x0 = xv[0, pl.ds(col, 16)]
                x1 = xv[1, pl.ds(col, 16)]
                x2 = xv[2, pl.ds(col, 16)]

                def l_body(l, _):
                    res_f = jnp.full((16,), jnp.left_shift(16, l),
                                     dtype=jnp.int32).astype(jnp.float32)
                    pos0 = x0 * res_f
                    pos1 = x1 * res_f
                    pos2 = x2 * res_f
                    p0 = pos0.astype(jnp.int32)
                    p1 = pos1.astype(jnp.int32)
                    p2 = pos2.astype(jnp.int32)
                    w0 = pos0 - p0.astype(jnp.float32)
                    w1 = pos1 - p1.astype(jnp.float32)
                    w2 = pos2 - p2.astype(jnp.float32)
                    h1 = p1 * P1
                    h2 = p2 * P2
                    b0 = p0 + 1
                    b1 = h1 + P1
                    b2 = h2 + P2
                    txy = (p0 ^ h1, b0 ^ h1, p0 ^ b1, b0 ^ b1)
                    u0 = 1.0 - w0
                    u1 = 1.0 - w1
                    u2 = 1.0 - w2
                    axy = (u0 * u1, w0 * u1, u0 * w1, w0 * w1)
                    lbase = jnp.full((16,), jnp.left_shift(l, LOG2_T + 1),
                                     dtype=jnp.int32)
                    rbase = l * 8 * C + col
                    for corner in range(8):
                        o2 = (corner >> 2) & 1
                        hz = b2 if o2 else h2
                        wz = w2 if o2 else u2
                        h = (txy[corner & 3] ^ hz) & HASH_MASK
                        e0 = h + (h & BLK_MASK) + lbase
                        flat = rbase + corner * C
                        cole = iota2 + flat * 2
                        plsc.store_scatter(idxb, [cole], e0)
                        plsc.store_scatter(idxb, [cole + 1], e0 + 128)
                        wb[pl.ds(flat, 16)] = axy[corner & 3] * wz
                    return ()

                lax.fori_loop(0, N_LEVELS, l_body, (), unroll=False)
                return ()

            lax.fori_loop(0, NG, g_body, (), unroll=False)

        def fire(idxb, fb, sem):
            pltpu.async_copy(tab_hbm.at[idxb], fb, sem)

        def drain(idxb, fb, sem):
            pltpu.make_async_copy(tab_hbm.at[idxb], fb, sem).wait()

        def phase_b(ci, wb, fb):
            def acc_body(k, _):
                l = lax.shift_right_logical(k, 2)
                g = jnp.bitwise_and(k, NG - 1)
                col = g * 16
                pidx = iota16 + col
                rbase = l * 8 * C + col
                acc0 = _splat_f(0.0)
                acc1 = _splat_f(0.0)
                for corner in range(8):
                    flat = rbase + corner * C
                    w = wb[pl.ds(flat, 16)]
                    fvec2 = iota2 + flat * 2
                    acc0 = acc0 + plsc.load_gather(fb, [fvec2]) * w
                    acc1 = acc1 + plsc.load_gather(fb, [fvec2 + 1]) * w
                dest0 = pidx * OUT_W + l * 2
                plsc.store_scatter(encv, [dest0], acc0)
                plsc.store_scatter(encv, [dest0 + 1], acc1)
                return ()

            lax.fori_loop(0, N_LEVELS * NG, acc_body, (), unroll=False)
            base = tile_base + ci * C
            pltpu.sync_copy(encv, out_hbm.at[pl.ds(base * OUT_W, C * OUT_W)])

        phase_a(0, idx0, wb0)
        fire(idx0, fb0, sem0)

        def pipe_body(k, _):
            ca = 2 * k
            phase_a(ca + 1, idx1, wb1)
            fire(idx1, fb1, sem1)
            drain(idx0, fb0, sem0)
            phase_b(ca, wb0, fb0)

            @pl.when(k < nchunk // 2 - 1)
            def _():
                phase_a(ca + 2, idx0, wb0)
                fire(idx0, fb0, sem0)

            drain(idx1, fb1, sem1)
            phase_b(ca + 1, wb1, fb1)
            return ()

        lax.fori_loop(0, nchunk // 2, pipe_body, (), unroll=False)

    return enc_kernel


_ENCODER = _make_encoder()


def kernel(x, table, bounding_box):
    bb_min = bounding_box[:3]
    bb_max = bounding_box[3:]
    xn = (x - bb_min) / (bb_max - bb_min)
    mask = ((xn > 0) & (xn < 1)).all(axis=-1)
    xnt = xn.T
    tab_lin = table.reshape(N_LEVELS, T // 128, 128, N_FEATS)
    tab_lin = tab_lin.transpose(0, 1, 3, 2).reshape(N_LEVELS * T * N_FEATS)
    enc = _ENCODER(xnt, tab_lin).reshape(N_PTS, OUT_W)
    return (enc, mask)


# speedup vs baseline: 7.2956x; 1.4628x over previous
import functools

import jax
import jax.numpy as jnp
import numpy as np
from jax import lax
from jax.experimental import pallas as pl
from jax.experimental.pallas import tpu as pltpu
from jax.experimental.pallas import tpu_sc as plsc

N_LEVELS = 16
N_FEATS = 2
LOG2_T = 19
T = 2 ** LOG2_T
IN_DIM = 3
HASH_MASK = np.int32(T - 1)
P1 = np.int32(-1640531535)
P2 = np.int32(805459861)

N_PTS = 262144
NW = 32
C = 32
NG = C // 16
NI = N_LEVELS * 8
CHUNK_IDX = NI * C
OUT_W = N_LEVELS * N_FEATS
TOT = N_LEVELS * T * N_FEATS
PACK_W = 8
SUP_TOT = TOT // PACK_W
RP_CHUNK = 4096
RP_SHARE = TOT // 16


def _splat_i(v):
    return jnp.full((16,), v, dtype=jnp.int32)


def _splat_f(v):
    return jnp.full((16,), v, dtype=jnp.float32)


def _make_encoder(n_pts=N_PTS, interpret=False):
    pts_per_tile = n_pts // NW
    nchunk = pts_per_tile // C
    mesh = plsc.VectorSubcoreMesh(core_axis_name="c", subcore_axis_name="s")

    @functools.partial(
        pl.kernel,
        mesh=mesh,
        compiler_params=pltpu.CompilerParams(needs_layout_passes=False,
                                             use_tc_tiling_on_sc=False),
        interpret=interpret,
        out_type=[jax.ShapeDtypeStruct((n_pts * OUT_W,), jnp.float32),
                  jax.ShapeDtypeStruct((2 * SUP_TOT, PACK_W), jnp.float32)],
        scratch_types=[
            pltpu.VMEM((IN_DIM, C), jnp.float32),
            pltpu.VMEM((CHUNK_IDX,), jnp.int32),
            pltpu.VMEM((CHUNK_IDX,), jnp.int32),
            pltpu.VMEM((CHUNK_IDX,), jnp.int32),
            pltpu.VMEM((CHUNK_IDX,), jnp.int32),
            pltpu.VMEM((CHUNK_IDX,), jnp.float32),
            pltpu.VMEM((CHUNK_IDX,), jnp.float32),
            pltpu.VMEM((CHUNK_IDX, PACK_W), jnp.float32),
            pltpu.VMEM((CHUNK_IDX, PACK_W), jnp.float32),
            pltpu.VMEM((C * OUT_W,), jnp.float32),
            pltpu.VMEM((RP_CHUNK,), jnp.float32),
            pltpu.VMEM((RP_CHUNK // PACK_W, PACK_W), jnp.float32),
            pltpu.SemaphoreType.DMA,
            pltpu.SemaphoreType.DMA,
        ],
    )
    def enc_kernel(xnt_hbm, tab_hbm, out_hbm, pack_hbm, xv, idx0, idx1,
                   col0, col1, wb0, wb1, fb0, fb1, encv, natv, pstage,
                   sem0, sem1):
        wid = lax.axis_index("s") * 2 + lax.axis_index("c")
        tile_base = wid * pts_per_tile
        iota16 = lax.iota(jnp.int32, 16)
        iota2 = iota16 * 2
        my_core = lax.axis_index("c")
        sub = lax.axis_index("s")
        core_sup = my_core * SUP_TOT

        def rp_body(it, _):
            off = sub * RP_SHARE + it * RP_CHUNK
            pltpu.sync_copy(tab_hbm.at[pl.ds(off, RP_CHUNK)], natv)
            for grp in range(RP_CHUNK // 256):
                gbase = grp * 256
                for v in range(8):
                    f0 = natv[pl.ds(gbase + v * 16, 16)]
                    f1 = natv[pl.ds(gbase + 128 + v * 16, 16)]
                    gflat = iota2 + (gbase + v * 32)
                    rowv = lax.shift_right_logical(gflat, 3)
                    colv = jnp.bitwise_and(gflat, 7)
                    plsc.store_scatter(pstage, [rowv, colv], f0)
                    plsc.store_scatter(pstage, [rowv, colv + 1], f1)
            sup0 = core_sup + lax.shift_right_logical(off, 3)
            pltpu.sync_copy(pstage,
                            pack_hbm.at[pl.ds(sup0, RP_CHUNK // PACK_W), :])
            return ()

        lax.fori_loop(0, RP_SHARE // RP_CHUNK, rp_body, (), unroll=False)
        plsc.subcore_barrier()

        def phase_a(ci, idxb, colb, wb):
            base = tile_base + ci * C
            pltpu.sync_copy(xnt_hbm.at[:, pl.ds(base, C)], xv)

            def g_body(g, _):
                col = g * 16
                x0 = xv[0, pl.ds(col, 16)]
                x1 = xv[1, pl.ds(col, 16)]
                x2 = xv[2, pl.ds(col, 16)]

                def l_body(l, _):
                    sup_base = core_sup + jnp.left_shift(l, 20 - 3)
                    res_f = jnp.full((16,), jnp.left_shift(16, l),
                                     dtype=jnp.int32).astype(jnp.float32)
                    pos0 = x0 * res_f
                    pos1 = x1 * res_f
                    pos2 = x2 * res_f
                    p0 = pos0.astype(jnp.int32)
                    p1 = pos1.astype(jnp.int32)
                    p2 = pos2.astype(jnp.int32)
                    w0 = pos0 - p0.astype(jnp.float32)
                    w1 = pos1 - p1.astype(jnp.float32)
                    w2 = pos2 - p2.astype(jnp.float32)
                    h1 = p1 * P1
                    h2 = p2 * P2
                    b0 = p0 + 1
                    b1 = h1 + P1
                    b2 = h2 + P2
                    txy = (p0 ^ h1, b0 ^ h1, p0 ^ b1, b0 ^ b1)
                    u0 = 1.0 - w0
                    u1 = 1.0 - w1
                    u2 = 1.0 - w2
                    axy = (u0 * u1, w0 * u1, u0 * w1, w0 * w1)
                    supv = jnp.full((16,), sup_base, dtype=jnp.int32)
                    rbase = l * 8 * C + col
                    for corner in range(8):
                        o2 = (corner >> 2) & 1
                        hz = b2 if o2 else h2
                        wz = w2 if o2 else u2
                        h = (txy[corner & 3] ^ hz) & HASH_MASK
                        flat = rbase + corner * C
                        idxb[pl.ds(flat, 16)] = (
                            supv + lax.shift_right_logical(h, 2))
                        colb[pl.ds(flat, 16)] = jnp.bitwise_and(h, 3) * 2
                        wb[pl.ds(flat, 16)] = axy[corner & 3] * wz
                    return ()

                lax.fori_loop(0, N_LEVELS, l_body, (), unroll=False)
                return ()

            lax.fori_loop(0, NG, g_body, (), unroll=False)

        def fire(idxb, fb, sem):
            pltpu.async_copy(pack_hbm.at[idxb], fb, sem)

        def drain(idxb, fb, sem):
            pltpu.make_async_copy(pack_hbm.at[idxb], fb, sem).wait()

        def phase_b(ci, colb, wb, fb):
            def acc_body(k, _):
                l = lax.shift_right_logical(k, 1)
                g = jnp.bitwise_and(k, NG - 1)
                col = g * 16
                pidx = iota16 + col
                rbase = l * 8 * C + col
                acc0 = _splat_f(0.0)
                acc1 = _splat_f(0.0)
                for corner in range(8):
                    flat = rbase + corner * C
                    w = wb[pl.ds(flat, 16)]
                    cv = colb[pl.ds(flat, 16)]
                    rowv = iota16 + flat
                    acc0 = acc0 + plsc.load_gather(fb, [rowv, cv]) * w
                    acc1 = acc1 + plsc.load_gather(fb, [rowv, cv + 1]) * w
                dest0 = pidx * OUT_W + l * 2
                plsc.store_scatter(encv, [dest0], acc0)
                plsc.store_scatter(encv, [dest0 + 1], acc1)
                return ()

            lax.fori_loop(0, N_LEVELS * NG, acc_body, (), unroll=False)
            base = tile_base + ci * C
            pltpu.sync_copy(encv, out_hbm.at[pl.ds(base * OUT_W, C * OUT_W)])

        phase_a(0, idx0, col0, wb0)
        fire(idx0, fb0, sem0)

        def pipe_body(k, _):
            ca = 2 * k
            phase_a(ca + 1, idx1, col1, wb1)
            fire(idx1, fb1, sem1)
            drain(idx0, fb0, sem0)
            phase_b(ca, col0, wb0, fb0)

            @pl.when(k < nchunk // 2 - 1)
            def _():
                phase_a(ca + 2, idx0, col0, wb0)
                fire(idx0, fb0, sem0)

            drain(idx1, fb1, sem1)
            phase_b(ca + 1, col1, wb1, fb1)
            return ()

        lax.fori_loop(0, nchunk // 2, pipe_body, (), unroll=False)

    return enc_kernel


_ENCODER = _make_encoder()


def kernel(x, table, bounding_box):
    bb_min = bounding_box[:3]
    bb_max = bounding_box[3:]
    xn = (x - bb_min) / (bb_max - bb_min)
    mask = ((xn > 0) & (xn < 1)).all(axis=-1)
    xnt = xn.T
    tab_lin = table.reshape(N_LEVELS, T // 128, 128, N_FEATS)
    tab_lin = tab_lin.transpose(0, 1, 3, 2).reshape(TOT)
    enc, _pack = _ENCODER(xnt, tab_lin)
    enc = enc.reshape(N_PTS, OUT_W)
    return (enc, mask)


# speedup vs baseline: 8.0200x; 1.0993x over previous
import functools

import jax
import jax.numpy as jnp
import numpy as np
from jax import lax
from jax.experimental import pallas as pl
from jax.experimental.pallas import tpu as pltpu
from jax.experimental.pallas import tpu_sc as plsc

N_LEVELS = 16
N_FEATS = 2
LOG2_T = 19
T = 2 ** LOG2_T
IN_DIM = 3
HASH_MASK = np.int32(T - 1)
P1 = np.int32(-1640531535)
P2 = np.int32(805459861)

N_PTS = 262144
NW = 32
C = 32
NG = C // 16
NI = N_LEVELS * 8
CHUNK_IDX = NI * C
OUT_W = N_LEVELS * N_FEATS
TOT = N_LEVELS * T * N_FEATS
PACK_W = 8
SUP_TOT = TOT // PACK_W
RP_CHUNK = 4096
RP_SHARE = TOT // 16


def _splat_i(v):
    return jnp.full((16,), v, dtype=jnp.int32)


def _splat_f(v):
    return jnp.full((16,), v, dtype=jnp.float32)


def _make_encoder(n_pts=N_PTS, interpret=False):
    pts_per_tile = n_pts // NW
    nchunk = pts_per_tile // C
    mesh = plsc.VectorSubcoreMesh(core_axis_name="c", subcore_axis_name="s")

    @functools.partial(
        pl.kernel,
        mesh=mesh,
        compiler_params=pltpu.CompilerParams(needs_layout_passes=False,
                                             use_tc_tiling_on_sc=False),
        interpret=interpret,
        out_type=[jax.ShapeDtypeStruct((n_pts * OUT_W,), jnp.float32),
                  jax.ShapeDtypeStruct((2 * SUP_TOT, PACK_W), jnp.float32)],
        scratch_types=[
            pltpu.VMEM((IN_DIM, C), jnp.float32),
            pltpu.VMEM((CHUNK_IDX,), jnp.int32),
            pltpu.VMEM((CHUNK_IDX,), jnp.int32),
            pltpu.VMEM((CHUNK_IDX,), jnp.int32),
            pltpu.VMEM((CHUNK_IDX,), jnp.int32),
            pltpu.VMEM((CHUNK_IDX,), jnp.float32),
            pltpu.VMEM((CHUNK_IDX,), jnp.float32),
            pltpu.VMEM((CHUNK_IDX, PACK_W), jnp.float32),
            pltpu.VMEM((CHUNK_IDX, PACK_W), jnp.float32),
            pltpu.VMEM((C * OUT_W,), jnp.float32),
            pltpu.VMEM((RP_CHUNK,), jnp.float32),
            pltpu.VMEM((RP_CHUNK,), jnp.float32),
            pltpu.VMEM((RP_CHUNK // PACK_W, PACK_W), jnp.float32),
            pltpu.VMEM((RP_CHUNK // PACK_W, PACK_W), jnp.float32),
            pltpu.SemaphoreType.DMA,
            pltpu.SemaphoreType.DMA,
            pltpu.SemaphoreType.DMA,
            pltpu.SemaphoreType.DMA,
            pltpu.SemaphoreType.DMA,
            pltpu.SemaphoreType.DMA,
        ],
    )
    def enc_kernel(xnt_hbm, tab_hbm, out_hbm, pack_hbm, xv, idx0, idx1,
                   col0, col1, wb0, wb1, fb0, fb1, encv, natv0, natv1,
                   pstage0, pstage1, sem0, sem1, ri0, ri1, ro0, ro1):
        wid = lax.axis_index("s") * 2 + lax.axis_index("c")
        tile_base = wid * pts_per_tile
        iota16 = lax.iota(jnp.int32, 16)
        iota2 = iota16 * 2
        my_core = lax.axis_index("c")
        sub = lax.axis_index("s")
        core_sup = my_core * SUP_TOT

        rp_n = RP_SHARE // RP_CHUNK

        def rp_off(it):
            return sub * RP_SHARE + it * RP_CHUNK

        def rp_in(it, natv, ri):
            pltpu.async_copy(tab_hbm.at[pl.ds(rp_off(it), RP_CHUNK)],
                             natv, ri)

        def rp_half(k, it, natv, pstage, ri, ro):
            off = rp_off(it)
            pltpu.make_async_copy(tab_hbm.at[pl.ds(off, RP_CHUNK)],
                                  natv, ri).wait()
            sup0 = core_sup + lax.shift_right_logical(off, 3)

            @pl.when(k > 0)
            def _():
                pltpu.make_async_copy(
                    pstage, pack_hbm.at[pl.ds(sup0, RP_CHUNK // PACK_W), :],
                    ro).wait()

            for grp in range(RP_CHUNK // 256):
                gbase = grp * 256
                for v in range(8):
                    f0 = natv[pl.ds(gbase + v * 16, 16)]
                    f1 = natv[pl.ds(gbase + 128 + v * 16, 16)]
                    gflat = iota2 + (gbase + v * 32)
                    rowv = lax.shift_right_logical(gflat, 3)
                    colv = jnp.bitwise_and(gflat, 7)
                    plsc.store_scatter(pstage, [rowv, colv], f0)
                    plsc.store_scatter(pstage, [rowv, colv + 1], f1)
            pltpu.async_copy(pstage,
                             pack_hbm.at[pl.ds(sup0, RP_CHUNK // PACK_W), :],
                             ro)

            @pl.when(it + 2 < rp_n)
            def _():
                rp_in(it + 2, natv, ri)

            return ()

        rp_in(0, natv0, ri0)
        rp_in(1, natv1, ri1)

        def rp_body(k, _):
            rp_half(k, 2 * k, natv0, pstage0, ri0, ro0)
            rp_half(k, 2 * k + 1, natv1, pstage1, ri1, ro1)
            return ()

        lax.fori_loop(0, rp_n // 2, rp_body, (), unroll=False)
        pltpu.make_async_copy(
            pstage0, pack_hbm.at[pl.ds(core_sup, RP_CHUNK // PACK_W), :],
            ro0).wait()
        pltpu.make_async_copy(
            pstage1, pack_hbm.at[pl.ds(core_sup, RP_CHUNK // PACK_W), :],
            ro1).wait()
        plsc.subcore_barrier()

        def phase_a(ci, idxb, colb, wb):
            base = tile_base + ci * C
            pltpu.sync_copy(xnt_hbm.at[:, pl.ds(base, C)], xv)

            def g_body(g, _):
                col = g * 16
                x0 = xv[0, pl.ds(col, 16)]
                x1 = xv[1, pl.ds(col, 16)]
                x2 = xv[2, pl.ds(col, 16)]

                def l_body(l, _):
                    sup_base = core_sup + jnp.left_shift(l, 20 - 3)
                    res_f = jnp.full((16,), jnp.left_shift(16, l),
                                     dtype=jnp.int32).astype(jnp.float32)
                    pos0 = x0 * res_f
                    pos1 = x1 * res_f
                    pos2 = x2 * res_f
                    p0 = pos0.astype(jnp.int32)
                    p1 = pos1.astype(jnp.int32)
                    p2 = pos2.astype(jnp.int32)
                    w0 = pos0 - p0.astype(jnp.float32)
                    w1 = pos1 - p1.astype(jnp.float32)
                    w2 = pos2 - p2.astype(jnp.float32)
                    h1 = p1 * P1
                    h2 = p2 * P2
                    b0 = p0 + 1
                    b1 = h1 + P1
                    b2 = h2 + P2
                    txy = (p0 ^ h1, b0 ^ h1, p0 ^ b1, b0 ^ b1)
                    u0 = 1.0 - w0
                    u1 = 1.0 - w1
                    u2 = 1.0 - w2
                    axy = (u0 * u1, w0 * u1, u0 * w1, w0 * w1)
                    supv = jnp.full((16,), sup_base, dtype=jnp.int32)
                    rbase = l * 8 * C + col
                    for corner in range(8):
                        o2 = (corner >> 2) & 1
                        hz = b2 if o2 else h2
                        wz = w2 if o2 else u2
                        h = (txy[corner & 3] ^ hz) & HASH_MASK
                        flat = rbase + corner * C
                        idxb[pl.ds(flat, 16)] = (
                            supv + lax.shift_right_logical(h, 2))
                        colb[pl.ds(flat, 16)] = jnp.bitwise_and(h, 3) * 2
                        wb[pl.ds(flat, 16)] = axy[corner & 3] * wz
                    return ()

                lax.fori_loop(0, N_LEVELS, l_body, (), unroll=False)
                return ()

            lax.fori_loop(0, NG, g_body, (), unroll=False)

        def fire(idxb, fb, sem):
            pltpu.async_copy(pack_hbm.at[idxb], fb, sem)

        def drain(idxb, fb, sem):
            pltpu.make_async_copy(pack_hbm.at[idxb], fb, sem).wait()

        def phase_b(ci, colb, wb, fb):
            def acc_body(k, _):
                l = lax.shift_right_logical(k, 1)
                g = jnp.bitwise_and(k, NG - 1)
                col = g * 16
                pidx = iota16 + col
                rbase = l * 8 * C + col
                acc0 = _splat_f(0.0)
                acc1 = _splat_f(0.0)
                for corner in range(8):
                    flat = rbase + corner * C
                    w = wb[pl.ds(flat, 16)]
                    cv = colb[pl.ds(flat, 16)]
                    rowv = iota16 + flat
                    acc0 = acc0 + plsc.load_gather(fb, [rowv, cv]) * w
                    acc1 = acc1 + plsc.load_gather(fb, [rowv, cv + 1]) * w
                dest0 = pidx * OUT_W + l * 2
                plsc.store_scatter(encv, [dest0], acc0)
                plsc.store_scatter(encv, [dest0 + 1], acc1)
                return ()

            lax.fori_loop(0, N_LEVELS * NG, acc_body, (), unroll=False)
            base = tile_base + ci * C
            pltpu.sync_copy(encv, out_hbm.at[pl.ds(base * OUT_W, C * OUT_W)])

        phase_a(0, idx0, col0, wb0)
        fire(idx0, fb0, sem0)

        def pipe_body(k, _):
            ca = 2 * k
            phase_a(ca + 1, idx1, col1, wb1)
            fire(idx1, fb1, sem1)
            drain(idx0, fb0, sem0)
            phase_b(ca, col0, wb0, fb0)

            @pl.when(k < nchunk // 2 - 1)
            def _():
                phase_a(ca + 2, idx0, col0, wb0)
                fire(idx0, fb0, sem0)

            drain(idx1, fb1, sem1)
            phase_b(ca + 1, col1, wb1, fb1)
            return ()

        lax.fori_loop(0, nchunk // 2, pipe_body, (), unroll=False)

    return enc_kernel


_ENCODER = _make_encoder()


def kernel(x, table, bounding_box):
    bb_min = bounding_box[:3]
    bb_max = bounding_box[3:]
    xn = (x - bb_min) / (bb_max - bb_min)
    mask = ((xn > 0) & (xn < 1)).all(axis=-1)
    xnt = xn.T
    tab_lin = table.reshape(N_LEVELS, T // 128, 128, N_FEATS)
    tab_lin = tab_lin.transpose(0, 1, 3, 2).reshape(TOT)
    enc, _pack = _ENCODER(xnt, tab_lin)
    enc = enc.reshape(N_PTS, OUT_W)
    return (enc, mask)


# speedup vs baseline: 8.0590x; 1.0049x over previous
import functools

import jax
import jax.numpy as jnp
import numpy as np
from jax import lax
from jax.experimental import pallas as pl
from jax.experimental.pallas import tpu as pltpu
from jax.experimental.pallas import tpu_sc as plsc

N_LEVELS = 16
N_FEATS = 2
LOG2_T = 19
T = 2 ** LOG2_T
IN_DIM = 3
HASH_MASK = np.int32(T - 1)
P1 = np.int32(-1640531535)
P2 = np.int32(805459861)

N_PTS = 262144
NW = 32
C = 32
NG = C // 16
NI = N_LEVELS * 8
CHUNK_IDX = NI * C
OUT_W = N_LEVELS * N_FEATS
TOT = N_LEVELS * T * N_FEATS
PACK_W = 8
SUP_TOT = TOT // PACK_W
RP_CHUNK = 4096
RP_SHARE = TOT // 16


def _splat_i(v):
    return jnp.full((16,), v, dtype=jnp.int32)


def _splat_f(v):
    return jnp.full((16,), v, dtype=jnp.float32)


def _make_encoder(n_pts=N_PTS, interpret=False):
    pts_per_tile = n_pts // NW
    nchunk = pts_per_tile // C
    mesh = plsc.VectorSubcoreMesh(core_axis_name="c", subcore_axis_name="s")

    @functools.partial(
        pl.kernel,
        mesh=mesh,
        compiler_params=pltpu.CompilerParams(needs_layout_passes=False,
                                             use_tc_tiling_on_sc=False),
        interpret=interpret,
        out_type=[jax.ShapeDtypeStruct((n_pts * OUT_W,), jnp.float32),
                  jax.ShapeDtypeStruct((2 * SUP_TOT, PACK_W), jnp.float32)],
        scratch_types=[
            pltpu.VMEM((IN_DIM, C), jnp.float32),
            pltpu.VMEM((CHUNK_IDX,), jnp.int32),
            pltpu.VMEM((CHUNK_IDX,), jnp.int32),
            pltpu.VMEM((CHUNK_IDX,), jnp.int32),
            pltpu.VMEM((CHUNK_IDX,), jnp.int32),
            pltpu.VMEM((CHUNK_IDX,), jnp.float32),
            pltpu.VMEM((CHUNK_IDX,), jnp.float32),
            pltpu.VMEM((CHUNK_IDX, PACK_W), jnp.float32),
            pltpu.VMEM((CHUNK_IDX, PACK_W), jnp.float32),
            pltpu.VMEM((C * OUT_W,), jnp.float32),
            pltpu.VMEM((RP_CHUNK,), jnp.float32),
            pltpu.VMEM((RP_CHUNK,), jnp.float32),
            pltpu.VMEM((RP_CHUNK // PACK_W, PACK_W), jnp.float32),
            pltpu.VMEM((RP_CHUNK // PACK_W, PACK_W), jnp.float32),
            pltpu.SemaphoreType.DMA,
            pltpu.SemaphoreType.DMA,
            pltpu.SemaphoreType.DMA,
            pltpu.SemaphoreType.DMA,
            pltpu.SemaphoreType.DMA,
            pltpu.SemaphoreType.DMA,
        ],
    )
    def enc_kernel(xnt_hbm, tab_hbm, out_hbm, pack_hbm, xv, idx0, idx1,
                   col0, col1, wb0, wb1, fb0, fb1, encv, natv0, natv1,
                   pstage0, pstage1, sem0, sem1, ri0, ri1, ro0, ro1):
        wid = lax.axis_index("s") * 2 + lax.axis_index("c")
        tile_base = wid * pts_per_tile
        iota16 = lax.iota(jnp.int32, 16)
        iota2 = iota16 * 2
        my_core = lax.axis_index("c")
        sub = lax.axis_index("s")
        core_sup = my_core * SUP_TOT

        rp_n = RP_SHARE // RP_CHUNK

        def rp_off(it):
            return sub * RP_SHARE + it * RP_CHUNK

        def rp_in(it, natv, ri):
            pltpu.async_copy(tab_hbm.at[pl.ds(rp_off(it), RP_CHUNK)],
                             natv, ri)

        def rp_half(k, it, natv, pstage, ri, ro):
            off = rp_off(it)
            pltpu.make_async_copy(tab_hbm.at[pl.ds(off, RP_CHUNK)],
                                  natv, ri).wait()
            sup0 = core_sup + lax.shift_right_logical(off, 3)

            @pl.when(k > 0)
            def _():
                pltpu.make_async_copy(
                    pstage, pack_hbm.at[pl.ds(sup0, RP_CHUNK // PACK_W), :],
                    ro).wait()

            for grp in range(RP_CHUNK // 256):
                gbase = grp * 256
                for v in range(8):
                    f0 = natv[pl.ds(gbase + v * 16, 16)]
                    f1 = natv[pl.ds(gbase + 128 + v * 16, 16)]
                    gflat = iota2 + (gbase + v * 32)
                    rowv = lax.shift_right_logical(gflat, 3)
                    colv = jnp.bitwise_and(gflat, 7)
                    plsc.store_scatter(pstage, [rowv, colv], f0)
                    plsc.store_scatter(pstage, [rowv, colv + 1], f1)
            pltpu.async_copy(pstage,
                             pack_hbm.at[pl.ds(sup0, RP_CHUNK // PACK_W), :],
                             ro)

            @pl.when(it + 2 < rp_n)
            def _():
                rp_in(it + 2, natv, ri)

            return ()

        rp_in(0, natv0, ri0)
        rp_in(1, natv1, ri1)

        def rp_body(k, _):
            rp_half(k, 2 * k, natv0, pstage0, ri0, ro0)
            rp_half(k, 2 * k + 1, natv1, pstage1, ri1, ro1)
            return ()

        lax.fori_loop(0, rp_n // 2, rp_body, (), unroll=False)
        pltpu.make_async_copy(
            pstage0, pack_hbm.at[pl.ds(core_sup, RP_CHUNK // PACK_W), :],
            ro0).wait()
        pltpu.make_async_copy(
            pstage1, pack_hbm.at[pl.ds(core_sup, RP_CHUNK // PACK_W), :],
            ro1).wait()
        plsc.subcore_barrier()

        def phase_a(ci, idxb, colb, wb):
            base = tile_base + ci * C
            pltpu.sync_copy(xnt_hbm.at[:, pl.ds(base, C)], xv)

            def g_body(g, _):
                col = g * 16
                x0 = xv[0, pl.ds(col, 16)]
                x1 = xv[1, pl.ds(col, 16)]
                x2 = xv[2, pl.ds(col, 16)]

                def l_body(l, _):
                    sup_base = core_sup + jnp.left_shift(l, 20 - 3)
                    res_f = jnp.full((16,), jnp.left_shift(16, l),
                                     dtype=jnp.int32).astype(jnp.float32)
                    pos0 = x0 * res_f
                    pos1 = x1 * res_f
                    pos2 = x2 * res_f
                    p0 = pos0.astype(jnp.int32)
                    p1 = pos1.astype(jnp.int32)
                    p2 = pos2.astype(jnp.int32)
                    w0 = pos0 - p0.astype(jnp.float32)
                    w1 = pos1 - p1.astype(jnp.float32)
                    w2 = pos2 - p2.astype(jnp.float32)
                    h1 = p1 * P1
                    h2 = p2 * P2
                    b0 = p0 + 1
                    b1 = h1 + P1
                    b2 = h2 + P2
                    txy = (p0 ^ h1, b0 ^ h1, p0 ^ b1, b0 ^ b1)
                    u0 = 1.0 - w0
                    u1 = 1.0 - w1
                    u2 = 1.0 - w2
                    axy = (u0 * u1, w0 * u1, u0 * w1, w0 * w1)
                    supv = jnp.full((16,), sup_base, dtype=jnp.int32)
                    rbase = l * 8 * C + col
                    for corner in range(8):
                        o2 = (corner >> 2) & 1
                        hz = b2 if o2 else h2
                        wz = w2 if o2 else u2
                        h = (txy[corner & 3] ^ hz) & HASH_MASK
                        flat = rbase + corner * C
                        idxb[pl.ds(flat, 16)] = (
                            supv + lax.shift_right_logical(h, 2))
                        colb[pl.ds(flat, 16)] = jnp.bitwise_and(h, 3) * 2
                        wb[pl.ds(flat, 16)] = axy[corner & 3] * wz
                    return ()

                lax.fori_loop(0, N_LEVELS, l_body, (), unroll=False)
                return ()

            lax.fori_loop(0, NG, g_body, (), unroll=False)

        HALF = CHUNK_IDX // 2

        def fire(idxb, fb, sem):
            pltpu.async_copy(pack_hbm.at[idxb.at[pl.ds(0, HALF)]],
                             fb.at[pl.ds(0, HALF), :], sem)
            pltpu.async_copy(pack_hbm.at[idxb.at[pl.ds(HALF, HALF)]],
                             fb.at[pl.ds(HALF, HALF), :], sem)

        def drain(idxb, fb, sem):
            pltpu.make_async_copy(pack_hbm.at[idxb.at[pl.ds(0, HALF)]],
                                  fb.at[pl.ds(0, HALF), :], sem).wait()
            pltpu.make_async_copy(pack_hbm.at[idxb.at[pl.ds(HALF, HALF)]],
                                  fb.at[pl.ds(HALF, HALF), :], sem).wait()

        def phase_b(ci, colb, wb, fb):
            def acc_body(k, _):
                l = lax.shift_right_logical(k, 1)
                g = jnp.bitwise_and(k, NG - 1)
                col = g * 16
                pidx = iota16 + col
                rbase = l * 8 * C + col
                acc0 = _splat_f(0.0)
                acc1 = _splat_f(0.0)
                for corner in range(8):
                    flat = rbase + corner * C
                    w = wb[pl.ds(flat, 16)]
                    cv = colb[pl.ds(flat, 16)]
                    rowv = iota16 + flat
                    acc0 = acc0 + plsc.load_gather(fb, [rowv, cv]) * w
                    acc1 = acc1 + plsc.load_gather(fb, [rowv, cv + 1]) * w
                dest0 = pidx * OUT_W + l * 2
                plsc.store_scatter(encv, [dest0], acc0)
                plsc.store_scatter(encv, [dest0 + 1], acc1)
                return ()

            lax.fori_loop(0, N_LEVELS * NG, acc_body, (), unroll=False)
            base = tile_base + ci * C
            pltpu.sync_copy(encv, out_hbm.at[pl.ds(base * OUT_W, C * OUT_W)])

        phase_a(0, idx0, col0, wb0)
        fire(idx0, fb0, sem0)

        def pipe_body(k, _):
            ca = 2 * k
            phase_a(ca + 1, idx1, col1, wb1)
            fire(idx1, fb1, sem1)
            drain(idx0, fb0, sem0)
            phase_b(ca, col0, wb0, fb0)

            @pl.when(k < nchunk // 2 - 1)
            def _():
                phase_a(ca + 2, idx0, col0, wb0)
                fire(idx0, fb0, sem0)

            drain(idx1, fb1, sem1)
            phase_b(ca + 1, col1, wb1, fb1)
            return ()

        lax.fori_loop(0, nchunk // 2, pipe_body, (), unroll=False)

    return enc_kernel


_ENCODER = _make_encoder()


def kernel(x, table, bounding_box):
    bb_min = bounding_box[:3]
    bb_max = bounding_box[3:]
    xn = (x - bb_min) / (bb_max - bb_min)
    mask = ((xn > 0) & (xn < 1)).all(axis=-1)
    xnt = xn.T
    tab_lin = table.reshape(N_LEVELS, T // 128, 128, N_FEATS)
    tab_lin = tab_lin.transpose(0, 1, 3, 2).reshape(TOT)
    enc, _pack = _ENCODER(xnt, tab_lin)
    enc = enc.reshape(N_PTS, OUT_W)
    return (enc, mask)
